# baseline (speedup 1.0000x reference)
import functools

import jax
import jax.numpy as jnp
from jax import lax
from jax.experimental import pallas as pl
from jax.experimental.pallas import tpu as pltpu
from jax.experimental.pallas import tpu_sc as plsc

N = 100000
M = 800000
D = 64

NW = 32
PER_W = M // NW
CHUNK = 1000
NCH = PER_W // CHUNK

_mesh = plsc.VectorSubcoreMesh(core_axis_name="c", subcore_axis_name="s")


@functools.partial(
    pl.kernel,
    mesh=_mesh,
    out_type=jax.ShapeDtypeStruct((M, D), jnp.float32),
    scratch_types=[
        pltpu.VMEM((CHUNK,), jnp.int32),
        pltpu.VMEM((CHUNK, D), jnp.float32),
        pltpu.SemaphoreType.DMA,
    ],
    compiler_params=pltpu.CompilerParams(use_tc_tiling_on_sc=False),
)
def _sc_gather(feats_hbm, idx_hbm, out_hbm, idx_v, rows_v, sem):
    c = lax.axis_index("c")
    s = lax.axis_index("s")
    wid = s * 2 + c
    base = wid * PER_W

    def body(j, _):
        off = pl.multiple_of(base + j * CHUNK, 8)
        pltpu.sync_copy(idx_hbm.at[pl.ds(off, CHUNK)], idx_v)
        pltpu.async_copy(feats_hbm.at[idx_v], rows_v, sem).wait()
        pltpu.sync_copy(rows_v, out_hbm.at[pl.ds(off, CHUNK)])
        return 0

    lax.fori_loop(0, NCH, body, 0)


def kernel(feats, idx):
    return _sc_gather(feats, idx.astype(jnp.int32))


# speedup vs baseline: 1.0294x; 1.0294x over previous
import functools

import jax
import jax.numpy as jnp
from jax import lax
from jax.experimental import pallas as pl
from jax.experimental.pallas import tpu as pltpu
from jax.experimental.pallas import tpu_sc as plsc

N = 100000
M = 800000
D = 64

NW = 32
PER_W = M // NW
CHUNK = 200
NB = 5
NCH = PER_W // CHUNK
NG = NCH // NB

_mesh = plsc.VectorSubcoreMesh(core_axis_name="c", subcore_axis_name="s")


@functools.partial(
    pl.kernel,
    mesh=_mesh,
    out_type=jax.ShapeDtypeStruct((M, D), jnp.float32),
    scratch_types=[
        pltpu.VMEM((PER_W,), jnp.int32),
        *[pltpu.VMEM((CHUNK, D), jnp.float32) for _ in range(NB)],
        *[pltpu.SemaphoreType.DMA for _ in range(2 * NB)],
    ],
    compiler_params=pltpu.CompilerParams(use_tc_tiling_on_sc=False),
)
def _sc_gather(feats_hbm, idx_hbm, out_hbm, idx_v, *bufs):
    rows = bufs[:NB]
    gsem = bufs[NB : 2 * NB]
    ssem = bufs[2 * NB :]

    c = lax.axis_index("c")
    s = lax.axis_index("s")
    wid = s * 2 + c
    base = pl.multiple_of(wid * PER_W, 8)
    pltpu.sync_copy(idx_hbm.at[pl.ds(base, PER_W)], idx_v)

    def group(g, _):
        gh = []
        for b in range(NB):
            off = pl.multiple_of((g * NB + b) * CHUNK, 8)
            gh.append(
                pltpu.async_copy(
                    feats_hbm.at[idx_v.at[pl.ds(off, CHUNK)]], rows[b], gsem[b]
                )
            )
        sh = []
        for b in range(NB):
            off = pl.multiple_of(base + (g * NB + b) * CHUNK, 8)
            gh[b].wait()
            sh.append(
                pltpu.async_copy(rows[b], out_hbm.at[pl.ds(off, CHUNK)], ssem[b])
            )
        for h in sh:
            h.wait()
        return 0

    lax.fori_loop(0, NG, group, 0)


def kernel(feats, idx):
    return _sc_gather(feats, idx.astype(jnp.int32))
